# baseline (speedup 1.0000x reference)
import jax
import jax.numpy as jnp
from jax.experimental import pallas as pl


def _seg_softmax(e, seg, n):
    m = jax.ops.segment_max(e, seg, num_segments=n)
    m = jnp.where(jnp.isfinite(m), m, 0.0)
    ex = jnp.exp(e - m[seg])
    s = jax.ops.segment_sum(ex, seg, num_segments=n)
    return ex / (s[seg] + 1e-16)


def _sage(x, src, dst, wl, bl, wr, n):
    agg = jax.ops.segment_sum(x[src], dst, num_segments=n)
    cnt = jax.ops.segment_sum(jnp.ones((src.shape[0],), jnp.float32), dst, num_segments=n)
    agg = agg / jnp.maximum(cnt, 1.0)[:, None]
    return agg @ wl + bl + x @ wr


def _gin(x, src, dst, w1, b1, w2, b2, n):
    agg = jax.ops.segment_sum(x[src], dst, num_segments=n)
    h = x + agg
    h = jax.nn.relu(h @ w1 + b1)
    h = jax.nn.relu(h @ w2 + b2)
    return h


def _gat(x, src, dst, w, a_s, a_d, b, heads, dh, n):
    loop = jnp.arange(n, dtype=src.dtype)
    s = jnp.concatenate([src, loop])
    d = jnp.concatenate([dst, loop])
    h = (x @ w).reshape(n, heads, dh)
    es = (h * a_s[None]).sum(-1)
    ed = (h * a_d[None]).sum(-1)
    e = jax.nn.leaky_relu(es[s] + ed[d], 0.2)
    alpha = _seg_softmax(e, d, n)
    out = jax.ops.segment_sum(h[s] * alpha[:, :, None], d, num_segments=n)
    return out.reshape(n, heads * dh) + b


def _bn(h):
    mu = h.mean(0)
    var = h.var(0)
    return (h - mu) / jnp.sqrt(var + 1e-5)


def _pair_kernel(x1_ref, x2_ref, w_ref, b_ref, o_ref):
    o_ref[...] = (x1_ref[...] * x2_ref[...]) @ w_ref[...] + b_ref[...]


def kernel(x, params, edge_index, train_edge_id):
    p = params
    n = x.shape[0]
    src, dst = edge_index[0], edge_index[1]
    xs = x[:, :, :16]
    h = jax.nn.relu(xs @ p['pW1'] + p['pb1'])
    h = jax.nn.relu(h @ p['pW2'] + p['pb2'])
    h = h @ p['pW3'] + p['pb3']
    feat = h.max(axis=1)
    hs = jax.nn.relu(_sage(feat, src, dst, p['s1_wl'], p['s1_bl'], p['s1_wr'], n))
    hs = _sage(hs, src, dst, p['s2_wl'], p['s2_bl'], p['s2_wr'], n)
    hg = _gin(feat, src, dst, p['g1_w1'], p['g1_b1'], p['g1_w2'], p['g1_b2'], n)
    hg = jax.nn.relu(hg)
    hg = _gin(hg, src, dst, p['g2_w1'], p['g2_b1'], p['g2_w2'], p['g2_b2'], n)
    hg = jax.nn.relu(hg)
    hg = hg @ p['gin_lw'] + p['gin_lb']
    ha = jax.nn.relu(_gat(feat, src, dst, p['a1_w'], p['a1_as'], p['a1_ad'], p['a1_b'], 8, 10, n))
    ha = _gat(ha, src, dst, p['a2_w'], p['a2_as'], p['a2_ad'], p['a2_b'], 1, 512, n)
    fused = p['fw'][0] * hs + p['fw'][1] * hg + p['fw'][2] * ha
    z = _bn(fused)
    z = jax.nn.relu(z @ p['l1_w'] + p['l1_b'])
    z = z @ p['l2_w'] + p['l2_b']
    node_id = edge_index[:, train_edge_id]
    x1 = z[node_id[0]]
    x2 = z[node_id[1]]
    out = pl.pallas_call(
        _pair_kernel,
        out_shape=jax.ShapeDtypeStruct((x1.shape[0], 7), jnp.float32),
    )(x1, x2, p['fc_w'], p['fc_b'])
    return out


# speedup vs baseline: 8.8272x; 8.8272x over previous
import functools

import jax
import jax.numpy as jnp
from jax import lax
from jax.experimental import pallas as pl
from jax.experimental.pallas import tpu as pltpu
from jax.experimental.pallas import tpu_sc as plsc

_N = 10000
_E = 160000
_NC, _NS = 2, 16
_NW = _NC * _NS
_NPAD = 10240
_RPS = _NPAD // _NS


def _chunk(per_w, d):
    for c in (256, 200, 128, 40, 8):
        if per_w % c == 0 and c * d * 4 <= 480 * 1024:
            return c
    return per_w


def _sc_mesh():
    return plsc.VectorSubcoreMesh(core_axis_name="c", subcore_axis_name="s")


def _sc_gather(table, idx):
    t_rows, d = table.shape
    e = idx.shape[0]
    per_w = e // _NW
    c = _chunk(per_w, d)
    n = per_w // c

    @functools.partial(
        pl.kernel, mesh=_sc_mesh(),
        out_type=jax.ShapeDtypeStruct((e, d), table.dtype),
        scratch_types=[
            pltpu.VMEM((c,), jnp.int32),
            pltpu.VMEM((c, d), table.dtype),
            pltpu.SemaphoreType.DMA,
        ],
    )
    def k(table_hbm, idx_hbm, out_hbm, idx_v, rows_v, sem):
        wid = lax.axis_index("s") * _NC + lax.axis_index("c")
        for j in range(n):
            base = wid * per_w + j * c
            pltpu.sync_copy(idx_hbm.at[pl.ds(base, c)], idx_v)
            pltpu.async_copy(table_hbm.at[idx_v], rows_v, sem).wait()
            pltpu.sync_copy(rows_v, out_hbm.at[pl.ds(base, c)])

    return k(table, idx)


def _sc_gss(table, src, dst):
    t_rows, d = table.shape
    e = src.shape[0]
    per_w = e // _NW
    c = _chunk(per_w, d)
    n = per_w // c
    zeros = jnp.zeros((_RPS, d), jnp.float32)

    @functools.partial(
        pl.kernel, mesh=_sc_mesh(),
        out_type=jax.ShapeDtypeStruct((_NC * _NPAD, d), jnp.float32),
        scratch_types=[
            pltpu.VMEM((c,), jnp.int32),
            pltpu.VMEM((c,), jnp.int32),
            pltpu.VMEM((c, d), jnp.float32),
            pltpu.VMEM_SHARED((_NPAD, d), jnp.float32),
            pltpu.SemaphoreType.DMA,
        ],
    )
    def k(table_hbm, src_hbm, dst_hbm, z_hbm, out_hbm,
          sidx_v, didx_v, rows_v, acc, sem):
        cid = lax.axis_index("c")
        sid = lax.axis_index("s")
        wid = sid * _NC + cid
        pltpu.sync_copy(z_hbm, acc.at[pl.ds(sid * _RPS, _RPS)])
        plsc.subcore_barrier()
        for j in range(n):
            base = wid * per_w + j * c
            pltpu.sync_copy(src_hbm.at[pl.ds(base, c)], sidx_v)
            pltpu.sync_copy(dst_hbm.at[pl.ds(base, c)], didx_v)
            pltpu.async_copy(table_hbm.at[sidx_v], rows_v, sem).wait()
            pltpu.sync_copy(rows_v, acc.at[didx_v], add=True)
        plsc.subcore_barrier()
        pltpu.sync_copy(acc.at[pl.ds(sid * _RPS, _RPS)],
                        out_hbm.at[pl.ds(cid * _NPAD + sid * _RPS, _RPS)])

    out = k(table, src, dst, zeros)
    return (out[:_N] + out[_NPAD:_NPAD + _N])


def _sc_scatter(msg, dst):
    e, d = msg.shape
    per_w = e // _NW
    c = _chunk(per_w, d)
    n = per_w // c
    zeros = jnp.zeros((_RPS, d), jnp.float32)

    @functools.partial(
        pl.kernel, mesh=_sc_mesh(),
        out_type=jax.ShapeDtypeStruct((_NC * _NPAD, d), jnp.float32),
        scratch_types=[
            pltpu.VMEM((c,), jnp.int32),
            pltpu.VMEM((c, d), jnp.float32),
            pltpu.VMEM_SHARED((_NPAD, d), jnp.float32),
            pltpu.SemaphoreType.DMA,
        ],
    )
    def k(msg_hbm, dst_hbm, z_hbm, out_hbm, didx_v, rows_v, acc, sem):
        cid = lax.axis_index("c")
        sid = lax.axis_index("s")
        wid = sid * _NC + cid
        pltpu.sync_copy(z_hbm, acc.at[pl.ds(sid * _RPS, _RPS)])
        plsc.subcore_barrier()
        for j in range(n):
            base = wid * per_w + j * c
            pltpu.sync_copy(dst_hbm.at[pl.ds(base, c)], didx_v)
            pltpu.sync_copy(msg_hbm.at[pl.ds(base, c)], rows_v)
            pltpu.sync_copy(rows_v, acc.at[didx_v], add=True)
        plsc.subcore_barrier()
        pltpu.sync_copy(acc.at[pl.ds(sid * _RPS, _RPS)],
                        out_hbm.at[pl.ds(cid * _NPAD + sid * _RPS, _RPS)])

    out = k(msg, dst, zeros)
    return (out[:_N] + out[_NPAD:_NPAD + _N])


def _mm(a, b):
    return jnp.dot(a, b, preferred_element_type=jnp.float32)


def _tc_pmlp(x, w1, b1, w2, b2, w3, b3):
    br = 400

    def body(x_ref, w1r, b1r, w2r, b2r, w3r, b3r, o_ref):
        xp = x_ref[...].reshape(br * 32, 16)
        h = jnp.maximum(_mm(xp, w1r[...]) + b1r[...], 0.0)
        h = jnp.maximum(_mm(h, w2r[...]) + b2r[...], 0.0)
        h = _mm(h, w3r[...]) + b3r[...]
        o_ref[...] = h.reshape(br, 32, 256).max(axis=1)

    return pl.pallas_call(
        body,
        grid=(_N // br,),
        in_specs=[
            pl.BlockSpec((br, 32, 16), lambda i: (i, 0, 0)),
            pl.BlockSpec((16, 64), lambda i: (0, 0)),
            pl.BlockSpec((1, 64), lambda i: (0, 0)),
            pl.BlockSpec((64, 128), lambda i: (0, 0)),
            pl.BlockSpec((1, 128), lambda i: (0, 0)),
            pl.BlockSpec((128, 256), lambda i: (0, 0)),
            pl.BlockSpec((1, 256), lambda i: (0, 0)),
        ],
        out_specs=pl.BlockSpec((br, 256), lambda i: (i, 0)),
        out_shape=jax.ShapeDtypeStruct((_N, 256), jnp.float32),
    )(x, w1, b1.reshape(1, -1), w2, b2.reshape(1, -1), w3, b3.reshape(1, -1))


def _tc_dense(x, w, b, act=None, br=2000):
    r, kdim = x.shape
    _, no = w.shape

    def body(x_ref, w_ref, b_ref, o_ref):
        y = _mm(x_ref[...], w_ref[...]) + b_ref[...]
        if act == "relu":
            y = jnp.maximum(y, 0.0)
        o_ref[...] = y

    return pl.pallas_call(
        body,
        grid=(r // br,),
        in_specs=[
            pl.BlockSpec((br, kdim), lambda i: (i, 0)),
            pl.BlockSpec((kdim, no), lambda i: (0, 0)),
            pl.BlockSpec((1, no), lambda i: (0, 0)),
        ],
        out_specs=pl.BlockSpec((br, no), lambda i: (i, 0)),
        out_shape=jax.ShapeDtypeStruct((r, no), jnp.float32),
    )(x, w, b.reshape(1, no))


def _tc_sage(agg, cnt, x, wl, bl, wr, act, br=2000):
    kdim = x.shape[1]
    no = wl.shape[1]

    def body(a_ref, c_ref, x_ref, wl_ref, bl_ref, wr_ref, o_ref):
        mean = a_ref[...] / jnp.maximum(c_ref[...][:, 0:1], 1.0)
        y = _mm(mean, wl_ref[...]) + bl_ref[...] + _mm(x_ref[...], wr_ref[...])
        if act == "relu":
            y = jnp.maximum(y, 0.0)
        o_ref[...] = y

    return pl.pallas_call(
        body,
        grid=(_N // br,),
        in_specs=[
            pl.BlockSpec((br, kdim), lambda i: (i, 0)),
            pl.BlockSpec((br, 128), lambda i: (i, 0)),
            pl.BlockSpec((br, kdim), lambda i: (i, 0)),
            pl.BlockSpec((kdim, no), lambda i: (0, 0)),
            pl.BlockSpec((1, no), lambda i: (0, 0)),
            pl.BlockSpec((kdim, no), lambda i: (0, 0)),
        ],
        out_specs=pl.BlockSpec((br, no), lambda i: (i, 0)),
        out_shape=jax.ShapeDtypeStruct((_N, no), jnp.float32),
    )(agg, cnt, x, wl, bl.reshape(1, no), wr)


def _tc_gin(x, agg, w1, b1, w2, b2, br=2000):
    kdim = x.shape[1]
    n1 = w1.shape[1]
    n2 = w2.shape[1]

    def body(x_ref, a_ref, w1r, b1r, w2r, b2r, o_ref):
        h = x_ref[...] + a_ref[...]
        h = jnp.maximum(_mm(h, w1r[...]) + b1r[...], 0.0)
        o_ref[...] = jnp.maximum(_mm(h, w2r[...]) + b2r[...], 0.0)

    return pl.pallas_call(
        body,
        grid=(_N // br,),
        in_specs=[
            pl.BlockSpec((br, kdim), lambda i: (i, 0)),
            pl.BlockSpec((br, kdim), lambda i: (i, 0)),
            pl.BlockSpec((kdim, n1), lambda i: (0, 0)),
            pl.BlockSpec((1, n1), lambda i: (0, 0)),
            pl.BlockSpec((n1, n2), lambda i: (0, 0)),
            pl.BlockSpec((1, n2), lambda i: (0, 0)),
        ],
        out_specs=pl.BlockSpec((br, n2), lambda i: (i, 0)),
        out_shape=jax.ShapeDtypeStruct((_N, n2), jnp.float32),
    )(x, agg, w1, b1.reshape(1, n1), w2, b2.reshape(1, n2))


def _tc_att(es_m, ed_m, h_m, heads, dh, br=4000):
    e, d = h_m.shape
    d_pad = ((d + 127) // 128) * 128

    def body(es_ref, ed_ref, h_ref, o_ref):
        t = es_ref[...] + ed_ref[...]
        w = jnp.exp(jnp.where(t > 0, t, 0.2 * t))
        wh = h_ref[...].reshape(br, heads, dh) * w[:, :heads].reshape(br, heads, 1)
        wh = wh.reshape(br, d)
        if d_pad > d:
            wh = jnp.concatenate(
                [wh, jnp.zeros((br, d_pad - d), jnp.float32)], axis=1)
        o_ref[...] = jnp.concatenate([wh, w], axis=1)

    return pl.pallas_call(
        body,
        grid=(e // br,),
        in_specs=[
            pl.BlockSpec((br, 128), lambda i: (i, 0)),
            pl.BlockSpec((br, 128), lambda i: (i, 0)),
            pl.BlockSpec((br, d), lambda i: (i, 0)),
        ],
        out_specs=pl.BlockSpec((br, d_pad + 128), lambda i: (i, 0)),
        out_shape=jax.ShapeDtypeStruct((e, d_pad + 128), jnp.float32),
    )(es_m, ed_m, h_m)


def _tc_gatfin(num, den, es, ed, h, b, heads, dh, act, br=2000):
    d = heads * dh

    def body(n_ref, d_ref, es_ref, ed_ref, h_ref, b_ref, o_ref):
        t = es_ref[...][:, :heads] + ed_ref[...][:, :heads]
        sw = jnp.exp(jnp.where(t > 0, t, 0.2 * t))
        dt = d_ref[...][:, :heads] + sw + 1e-16
        out = (n_ref[...].reshape(br, heads, dh)
               + sw.reshape(br, heads, 1) * h_ref[...].reshape(br, heads, dh))
        out = out / dt.reshape(br, heads, 1)
        y = out.reshape(br, d) + b_ref[...]
        if act == "relu":
            y = jnp.maximum(y, 0.0)
        o_ref[...] = y

    return pl.pallas_call(
        body,
        grid=(_N // br,),
        in_specs=[
            pl.BlockSpec((br, d), lambda i: (i, 0)),
            pl.BlockSpec((br, 128), lambda i: (i, 0)),
            pl.BlockSpec((br, 128), lambda i: (i, 0)),
            pl.BlockSpec((br, 128), lambda i: (i, 0)),
            pl.BlockSpec((br, d), lambda i: (i, 0)),
            pl.BlockSpec((1, d), lambda i: (0, 0)),
        ],
        out_specs=pl.BlockSpec((br, d), lambda i: (i, 0)),
        out_shape=jax.ShapeDtypeStruct((_N, d), jnp.float32),
    )(num, den, es, ed, h, b.reshape(1, d))


def _tc_fuse(hs, hg, ha, fw, br=2000):
    def body(hs_ref, hg_ref, ha_ref, fw_ref, f_ref, s_ref):
        f = (fw_ref[0, 0] * hs_ref[...] + fw_ref[0, 1] * hg_ref[...]
             + fw_ref[0, 2] * ha_ref[...])
        f_ref[...] = f

        @pl.when(pl.program_id(0) == 0)
        def _():
            s_ref[...] = jnp.zeros_like(s_ref)

        s_ref[0:1, :] += jnp.sum(f, axis=0, keepdims=True)

    return pl.pallas_call(
        body,
        grid=(_N // br,),
        in_specs=[
            pl.BlockSpec((br, 512), lambda i: (i, 0)),
            pl.BlockSpec((br, 512), lambda i: (i, 0)),
            pl.BlockSpec((br, 512), lambda i: (i, 0)),
            pl.BlockSpec((8, 128), lambda i: (0, 0)),
        ],
        out_specs=[
            pl.BlockSpec((br, 512), lambda i: (i, 0)),
            pl.BlockSpec((8, 512), lambda i: (0, 0)),
        ],
        out_shape=[
            jax.ShapeDtypeStruct((_N, 512), jnp.float32),
            jax.ShapeDtypeStruct((8, 512), jnp.float32),
        ],
    )(hs, hg, ha, fw)


def _tc_var(fused, s, br=2000):
    def body(f_ref, s_ref, ss_ref):
        d = f_ref[...] - s_ref[0:1, :] / float(_N)

        @pl.when(pl.program_id(0) == 0)
        def _():
            ss_ref[...] = jnp.zeros_like(ss_ref)

        ss_ref[0:1, :] += jnp.sum(d * d, axis=0, keepdims=True)

    return pl.pallas_call(
        body,
        grid=(_N // br,),
        in_specs=[
            pl.BlockSpec((br, 512), lambda i: (i, 0)),
            pl.BlockSpec((8, 512), lambda i: (0, 0)),
        ],
        out_specs=pl.BlockSpec((8, 512), lambda i: (0, 0)),
        out_shape=jax.ShapeDtypeStruct((8, 512), jnp.float32),
    )(fused, s)


def _tc_norm(fused, s, ss, w, b, br=2000):
    def body(f_ref, s_ref, ss_ref, w_ref, b_ref, o_ref):
        mu = s_ref[0:1, :] / float(_N)
        var = ss_ref[0:1, :] / float(_N)
        z = (f_ref[...] - mu) / jnp.sqrt(var + 1e-5)
        o_ref[...] = jnp.maximum(_mm(z, w_ref[...]) + b_ref[...], 0.0)

    return pl.pallas_call(
        body,
        grid=(_N // br,),
        in_specs=[
            pl.BlockSpec((br, 512), lambda i: (i, 0)),
            pl.BlockSpec((8, 512), lambda i: (0, 0)),
            pl.BlockSpec((8, 512), lambda i: (0, 0)),
            pl.BlockSpec((512, 512), lambda i: (0, 0)),
            pl.BlockSpec((1, 512), lambda i: (0, 0)),
        ],
        out_specs=pl.BlockSpec((br, 512), lambda i: (i, 0)),
        out_shape=jax.ShapeDtypeStruct((_N, 512), jnp.float32),
    )(fused, s, ss, w, b.reshape(1, 512))


def _tc_pair(x1, x2, w, b, br=2048):
    r = x1.shape[0]

    def body(x1_ref, x2_ref, w_ref, b_ref, o_ref):
        o_ref[...] = _mm(x1_ref[...] * x2_ref[...], w_ref[...]) + b_ref[...]

    return pl.pallas_call(
        body,
        grid=(r // br,),
        in_specs=[
            pl.BlockSpec((br, 512), lambda i: (i, 0)),
            pl.BlockSpec((br, 512), lambda i: (i, 0)),
            pl.BlockSpec((512, 7), lambda i: (0, 0)),
            pl.BlockSpec((1, 7), lambda i: (0, 0)),
        ],
        out_specs=pl.BlockSpec((br, 7), lambda i: (i, 0)),
        out_shape=jax.ShapeDtypeStruct((r, 7), jnp.float32),
    )(x1, x2, w, b.reshape(1, 7))


def _head_mix(a):
    heads, dh = a.shape
    eye = jnp.eye(heads, dtype=jnp.float32)
    mat = (a[:, :, None] * eye[:, None, :]).reshape(heads * dh, heads)
    return jnp.pad(mat, ((0, 0), (0, 128 - heads)))


def kernel(x, params, edge_index, train_edge_id):
    p = params
    src, dst = edge_index[0], edge_index[1]

    feat = _tc_pmlp(x, p['pW1'], p['pb1'], p['pW2'], p['pb2'], p['pW3'], p['pb3'])

    ones = jnp.ones((_N, 128), jnp.float32)
    cnt = _sc_gss(ones, src, dst)
    agg1 = jnp.concatenate(
        [_sc_gss(feat[:, :128], src, dst), _sc_gss(feat[:, 128:], src, dst)],
        axis=1)

    hs = _tc_sage(agg1, cnt, feat, p['s1_wl'], p['s1_bl'], p['s1_wr'], "relu")
    agg_s2 = _sc_gss(hs, src, dst)
    hs = _tc_sage(agg_s2, cnt, hs, p['s2_wl'], p['s2_bl'], p['s2_wr'], None)

    hg = _tc_gin(feat, agg1, p['g1_w1'], p['g1_b1'], p['g1_w2'], p['g1_b2'])
    agg_g2 = _sc_gss(hg, src, dst)
    hg = _tc_gin(hg, agg_g2, p['g2_w1'], p['g2_b1'], p['g2_w2'], p['g2_b2'])
    hg = _tc_dense(hg, p['gin_lw'], p['gin_lb'], None)

    z80 = jnp.zeros((80,), jnp.float32)
    z128 = jnp.zeros((128,), jnp.float32)
    h1 = _tc_dense(feat, p['a1_w'], z80, None)
    es1 = _tc_dense(h1, _head_mix(p['a1_as']), z128, None)
    ed1 = _tc_dense(h1, _head_mix(p['a1_ad']), z128, None)
    h1p = jnp.pad(h1, ((0, 0), (0, 48)))
    g1 = _sc_gather(jnp.concatenate([h1p, es1], axis=1), src)
    edm1 = _sc_gather(ed1, dst)
    pay1 = _tc_att(g1[:, 128:], edm1, g1[:, :80], 8, 10)
    num1 = _sc_scatter(pay1[:, :128], dst)
    den1 = _sc_scatter(pay1[:, 128:], dst)
    ha = _tc_gatfin(num1[:, :80], den1, es1, ed1, h1,
                    p['a1_b'], 8, 10, "relu")

    z512 = jnp.zeros((512,), jnp.float32)
    h2 = _tc_dense(ha, p['a2_w'], z512, None)
    es2 = _tc_dense(h2, _head_mix(p['a2_as']), z128, None)
    ed2 = _tc_dense(h2, _head_mix(p['a2_ad']), z128, None)
    hm2 = _sc_gather(h2, src)
    esm2 = _sc_gather(es2, src)
    edm2 = _sc_gather(ed2, dst)
    pay2 = _tc_att(esm2, edm2, hm2, 1, 512)
    num2 = jnp.concatenate(
        [_sc_scatter(pay2[:, k * 128:(k + 1) * 128], dst) for k in range(4)],
        axis=1)
    den2 = _sc_scatter(pay2[:, 512:], dst)
    ha = _tc_gatfin(num2, den2, es2, ed2, h2, p['a2_b'], 1, 512, None)

    fw = jnp.zeros((8, 128), jnp.float32).at[0, :3].set(p['fw'])
    fused, s = _tc_fuse(hs, hg, ha, fw)
    ss = _tc_var(fused, s)
    z = _tc_norm(fused, s, ss, p['l1_w'], p['l1_b'])
    z = _tc_dense(z, p['l2_w'], p['l2_b'], None)

    eit = jnp.pad(edge_index.T, ((0, 0), (0, 126)))
    nid = _sc_gather(eit, train_edge_id)
    x1 = _sc_gather(z, nid[:, 0])
    x2 = _sc_gather(z, nid[:, 1])
    return _tc_pair(x1, x2, p['fc_w'], p['fc_b'])
